# baseline (speedup 1.0000x reference)
import functools

import jax
import jax.numpy as jnp
from jax import lax
from jax.experimental import pallas as pl
from jax.experimental.pallas import tpu as pltpu
from jax.experimental.pallas import tpu_sc as plsc

N = 10000
D = 128
E = 320000

NC = 2
NS = 16
NW = NC * NS

NPAD = 10240
RPT = NPAD // NS
EPT = E // NW
CH = 100
NCHT = EPT // CH
NHALF = 2
NCHH = NCHT // NHALF

_mesh = plsc.VectorSubcoreMesh(
    core_axis_name="c", subcore_axis_name="s", num_cores=NC, num_subcores=NS)

def _fill2d(ref, nrows, ncols, val):
    vec = jnp.full((16,), val, jnp.float32)
    def row(r, carry):
        for j in range(ncols // 16):
            ref[r, pl.ds(j * 16, 16)] = vec
        return carry
    lax.fori_loop(0, nrows, row, 0)


def _deg_body(dst_hbm, out_hbm, dst_v, ones_v, deg_sh, sem):
    c = lax.axis_index("c")
    s = lax.axis_index("s")
    wid = s * NC + c

    _fill2d(ones_v, CH, D, 0.0)
    r0 = s * RPT
    for k in range(RPT // CH):
        pltpu.sync_copy(ones_v, deg_sh.at[pl.ds(r0 + k * CH, CH)])
    rem = RPT - (RPT // CH) * CH
    if rem:
        pltpu.sync_copy(ones_v.at[pl.ds(0, rem)],
                        deg_sh.at[pl.ds(r0 + (RPT // CH) * CH, rem)])
    _fill2d(ones_v, CH, D, 1.0)
    plsc.subcore_barrier()

    def issue(j, carry):
        pltpu.async_copy(ones_v, deg_sh.at[dst_v.at[j]], sem, add=True)
        return carry
    def drain(j, carry):
        pltpu.make_async_copy(ones_v, deg_sh.at[dst_v.at[0]], sem).wait()
        return carry
    for half in range(NHALF):
        pltpu.sync_copy(dst_hbm.at[wid * NHALF + half], dst_v)
        lax.fori_loop(0, NCHH, issue, 0)
        lax.fori_loop(0, NCHH, drain, 0)

    plsc.subcore_barrier()
    pltpu.sync_copy(deg_sh.at[pl.ds(r0, RPT)], out_hbm.at[c, pl.ds(r0, RPT)])


_deg_call = functools.partial(
    pl.kernel,
    out_type=jax.ShapeDtypeStruct((NC, NPAD, D), jnp.float32),
    mesh=_mesh,
    scratch_types=[
        pltpu.VMEM((NCHH, CH), jnp.int32),
        pltpu.VMEM((CH, D), jnp.float32),
        pltpu.VMEM_SHARED((NPAD, D), jnp.float32),
        pltpu.SemaphoreType.DMA,
    ],
)(_deg_body)


def _conv_body(g_hbm, src_hbm, dst_hbm, out_hbm,
               src_v, dst_v, rows_v, acc_sh, gsem0, gsem1, ssem):
    c = lax.axis_index("c")
    s = lax.axis_index("s")
    wid = s * NC + c

    zvec = jnp.zeros((16,), jnp.float32)
    def zrow(r, carry):
        for j in range(D // 16):
            rows_v[0, r, pl.ds(j * 16, 16)] = zvec
        return carry
    lax.fori_loop(0, CH, zrow, 0)
    r0 = s * RPT
    for k in range(RPT // CH):
        pltpu.sync_copy(rows_v.at[0], acc_sh.at[pl.ds(r0 + k * CH, CH)])
    rem = RPT - (RPT // CH) * CH
    if rem:
        pltpu.sync_copy(rows_v.at[0].at[pl.ds(0, rem)],
                        acc_sh.at[pl.ds(r0 + (RPT // CH) * CH, rem)])
    plsc.subcore_barrier()

    def body(j2, carry):
        j = j2 * 2
        cpb = pltpu.async_copy(g_hbm.at[src_v.at[j + 1]], rows_v.at[1], gsem1)
        pltpu.make_async_copy(g_hbm.at[src_v.at[j]], rows_v.at[0],
                              gsem0).wait()
        pltpu.async_copy(rows_v.at[0], acc_sh.at[dst_v.at[j]], ssem,
                         add=True).wait()
        @pl.when(j2 < NCHH // 2 - 1)
        def _():
            pltpu.async_copy(g_hbm.at[src_v.at[j + 2]], rows_v.at[0], gsem0)
        cpb.wait()
        pltpu.async_copy(rows_v.at[1], acc_sh.at[dst_v.at[j + 1]], ssem,
                         add=True).wait()
        return carry
    for half in range(NHALF):
        pltpu.sync_copy(src_hbm.at[wid * NHALF + half], src_v)
        pltpu.sync_copy(dst_hbm.at[wid * NHALF + half], dst_v)
        pltpu.async_copy(g_hbm.at[src_v.at[0]], rows_v.at[0], gsem0)
        lax.fori_loop(0, NCHH // 2, body, 0)

    plsc.subcore_barrier()
    pltpu.sync_copy(acc_sh.at[pl.ds(r0, RPT)], out_hbm.at[c, pl.ds(r0, RPT)])


_conv_call = functools.partial(
    pl.kernel,
    out_type=jax.ShapeDtypeStruct((NC, NPAD, D), jnp.float32),
    mesh=_mesh,
    scratch_types=[
        pltpu.VMEM((NCHH, CH), jnp.int32),
        pltpu.VMEM((NCHH, CH), jnp.int32),
        pltpu.VMEM((2, CH, D), jnp.float32),
        pltpu.VMEM_SHARED((NPAD, D), jnp.float32),
        pltpu.SemaphoreType.DMA,
        pltpu.SemaphoreType.DMA,
        pltpu.SemaphoreType.DMA,
    ],
)(_conv_body)


BR = 10240
GRID = NPAD // BR

def _row_spec(width):
    return pl.BlockSpec((BR, width), lambda i: (i, 0))

def _full_spec(shape):
    return pl.BlockSpec(shape, lambda i: (0,) * len(shape))


def _scale_body(x_ref, we_ref, be_ref, wg_ref, p0_ref, p1_ref,
                dinv_ref, g_ref):
    h = jnp.dot(x_ref[...], we_ref[...], preferred_element_type=jnp.float32)
    h = jnp.maximum(h + be_ref[...], 0.0)
    hr = jnp.dot(h, wg_ref[...], preferred_element_type=jnp.float32)
    deg = p0_ref[...] + p1_ref[...] + 1.0
    dinv = 1.0 / jnp.sqrt(deg)
    dinv_ref[...] = dinv
    g_ref[...] = dinv * hr


def _scale_call(x, W_enc, b_enc, W_gcn1, p0, p1):
    return pl.pallas_call(
        _scale_body,
        grid=(GRID,),
        in_specs=[_row_spec(D), _full_spec((D, D)), _full_spec((1, D)),
                  _full_spec((D, D)), _row_spec(D), _row_spec(D)],
        out_specs=[_row_spec(D), _row_spec(D)],
        out_shape=[jax.ShapeDtypeStruct((NPAD, D), jnp.float32),
                   jax.ShapeDtypeStruct((NPAD, D), jnp.float32)],
    )(x, W_enc, b_enc, W_gcn1, p0, p1)


def _mid_body(a0_ref, a1_ref, g_ref, dinv_ref, bg_ref, wu_ref, bu_ref,
              wn_ref, o_ref):
    dinv = dinv_ref[...]
    t = dinv * (a0_ref[...] + a1_ref[...] + g_ref[...]) + bg_ref[...]
    h = jnp.dot(t, wu_ref[...], preferred_element_type=jnp.float32)
    h = jnp.maximum(h + bu_ref[...], 0.0)
    o_ref[...] = dinv * jnp.dot(h, wn_ref[...],
                                preferred_element_type=jnp.float32)


def _mid_call(a0, a1, g, dinv, b_gcn, W_upd, b_upd, W_next):
    return pl.pallas_call(
        _mid_body,
        grid=(GRID,),
        in_specs=[_row_spec(D), _row_spec(D), _row_spec(D), _row_spec(D),
                  _full_spec((1, D)), _full_spec((D, D)), _full_spec((1, D)),
                  _full_spec((D, D))],
        out_specs=_row_spec(D),
        out_shape=jax.ShapeDtypeStruct((NPAD, D), jnp.float32),
    )(a0, a1, g, dinv, b_gcn, W_upd, b_upd, W_next)


def _out_body(a0_ref, a1_ref, g_ref, dinv_ref, bg_ref, wu_ref, bu_ref, o_ref):
    dinv = dinv_ref[...]
    t = dinv * (a0_ref[...] + a1_ref[...] + g_ref[...]) + bg_ref[...]
    h = jnp.dot(t, wu_ref[...], preferred_element_type=jnp.float32)
    o_ref[...] = jnp.maximum(h + bu_ref[...], 0.0)


def _out_call(a0, a1, g, dinv, b_gcn, W_upd, b_upd):
    return pl.pallas_call(
        _out_body,
        grid=(GRID,),
        in_specs=[_row_spec(D), _row_spec(D), _row_spec(D), _row_spec(D),
                  _full_spec((1, D)), _full_spec((D, D)), _full_spec((1, D))],
        out_specs=_row_spec(D),
        out_shape=jax.ShapeDtypeStruct((NPAD, D), jnp.float32),
    )(a0, a1, g, dinv, b_gcn, W_upd, b_upd)


def kernel(x, W_enc, b_enc, W_gcn1, b_gcn1, W_upd1, b_upd1,
           W_gcn2, b_gcn2, W_upd2, b_upd2, edge_index):
    src3d = edge_index[0].astype(jnp.int32).reshape(NW * NHALF, NCHH, CH)
    dst3d = edge_index[1].astype(jnp.int32).reshape(NW * NHALF, NCHH, CH)
    x_pad = jnp.zeros((NPAD, D), jnp.float32).at[:N].set(x)
    b_enc2 = b_enc.reshape(1, D)
    b_gcn1_2 = b_gcn1.reshape(1, D)
    b_upd1_2 = b_upd1.reshape(1, D)
    b_gcn2_2 = b_gcn2.reshape(1, D)
    b_upd2_2 = b_upd2.reshape(1, D)

    degp = _deg_call(dst3d)
    dinv, g1 = _scale_call(x_pad, W_enc, b_enc2, W_gcn1, degp[0], degp[1])
    acc1 = _conv_call(g1, src3d, dst3d)
    g2 = _mid_call(acc1[0], acc1[1], g1, dinv, b_gcn1_2, W_upd1, b_upd1_2,
                   W_gcn2)
    acc2 = _conv_call(g2, src3d, dst3d)
    out = _out_call(acc2[0], acc2[1], g2, dinv, b_gcn2_2, W_upd2, b_upd2_2)
    return out[:N]


# speedup vs baseline: 1.0155x; 1.0155x over previous
import functools

import jax
import jax.numpy as jnp
from jax import lax
from jax.experimental import pallas as pl
from jax.experimental.pallas import tpu as pltpu
from jax.experimental.pallas import tpu_sc as plsc

N = 10000
D = 128
E = 320000

NC = 2
NS = 16
NW = NC * NS

NPAD = 10240
RPT = NPAD // NS
EPT = E // NW
CH = 100
NCHT = EPT // CH
NHALF = 2
NCHH = NCHT // NHALF

_mesh = plsc.VectorSubcoreMesh(
    core_axis_name="c", subcore_axis_name="s", num_cores=NC, num_subcores=NS)

def _fill2d(ref, nrows, ncols, val):
    vec = jnp.full((16,), val, jnp.float32)
    def row(r, carry):
        for j in range(ncols // 16):
            ref[r, pl.ds(j * 16, 16)] = vec
        return carry
    lax.fori_loop(0, nrows, row, 0)


def _deg_body(dst_hbm, out_hbm, dst_v, ones_v, deg_sh, sem):
    c = lax.axis_index("c")
    s = lax.axis_index("s")
    wid = s * NC + c

    _fill2d(ones_v, CH, D, 0.0)
    r0 = s * RPT
    for k in range(RPT // CH):
        pltpu.sync_copy(ones_v, deg_sh.at[pl.ds(r0 + k * CH, CH)])
    rem = RPT - (RPT // CH) * CH
    if rem:
        pltpu.sync_copy(ones_v.at[pl.ds(0, rem)],
                        deg_sh.at[pl.ds(r0 + (RPT // CH) * CH, rem)])
    _fill2d(ones_v, CH, D, 1.0)
    plsc.subcore_barrier()

    def issue(j, carry):
        pltpu.async_copy(ones_v, deg_sh.at[dst_v.at[j]], sem, add=True)
        return carry
    def drain(j, carry):
        pltpu.make_async_copy(ones_v, deg_sh.at[dst_v.at[0]], sem).wait()
        return carry
    for half in range(NHALF):
        pltpu.sync_copy(dst_hbm.at[wid * NHALF + half], dst_v)
        lax.fori_loop(0, NCHH, issue, 0)
        lax.fori_loop(0, NCHH, drain, 0)

    plsc.subcore_barrier()
    pltpu.sync_copy(deg_sh.at[pl.ds(r0, RPT)], out_hbm.at[c, pl.ds(r0, RPT)])


_deg_call = functools.partial(
    pl.kernel,
    out_type=jax.ShapeDtypeStruct((NC, NPAD, D), jnp.float32),
    mesh=_mesh,
    scratch_types=[
        pltpu.VMEM((NCHH, CH), jnp.int32),
        pltpu.VMEM((CH, D), jnp.float32),
        pltpu.VMEM_SHARED((NPAD, D), jnp.float32),
        pltpu.SemaphoreType.DMA,
    ],
)(_deg_body)


def _conv_body(g_hbm, src_hbm, dst_hbm, out_hbm,
               src_v, dst_v, rows_v, acc_sh, gsem0, gsem1, ssem):
    c = lax.axis_index("c")
    s = lax.axis_index("s")
    wid = s * NC + c

    zvec = jnp.zeros((16,), jnp.float32)
    def zrow(r, carry):
        for j in range(D // 16):
            rows_v[0, r, pl.ds(j * 16, 16)] = zvec
        return carry
    lax.fori_loop(0, CH, zrow, 0)
    r0 = s * RPT
    for k in range(RPT // CH):
        pltpu.sync_copy(rows_v.at[0], acc_sh.at[pl.ds(r0 + k * CH, CH)])
    rem = RPT - (RPT // CH) * CH
    if rem:
        pltpu.sync_copy(rows_v.at[0].at[pl.ds(0, rem)],
                        acc_sh.at[pl.ds(r0 + (RPT // CH) * CH, rem)])
    plsc.subcore_barrier()

    def body(j2, carry):
        j = j2 * 2
        cpb = pltpu.async_copy(g_hbm.at[src_v.at[j + 1]], rows_v.at[1], gsem1)
        pltpu.make_async_copy(g_hbm.at[src_v.at[j]], rows_v.at[0],
                              gsem0).wait()
        pltpu.async_copy(rows_v.at[0], acc_sh.at[dst_v.at[j]], ssem,
                         add=True).wait()
        @pl.when(j2 < NCHH // 2 - 1)
        def _():
            pltpu.async_copy(g_hbm.at[src_v.at[j + 2]], rows_v.at[0], gsem0)
        cpb.wait()
        pltpu.async_copy(rows_v.at[1], acc_sh.at[dst_v.at[j + 1]], ssem,
                         add=True).wait()
        return carry
    for half in range(NHALF):
        pltpu.sync_copy(src_hbm.at[wid * NHALF + half], src_v)
        pltpu.sync_copy(dst_hbm.at[wid * NHALF + half], dst_v)
        pltpu.async_copy(g_hbm.at[src_v.at[0]], rows_v.at[0], gsem0)
        lax.fori_loop(0, NCHH // 2, body, 0)

    plsc.subcore_barrier()
    pltpu.sync_copy(acc_sh.at[pl.ds(r0, RPT)], out_hbm.at[c, pl.ds(r0, RPT)])


_conv_call = functools.partial(
    pl.kernel,
    out_type=jax.ShapeDtypeStruct((NC, NPAD, D), jnp.float32),
    mesh=_mesh,
    scratch_types=[
        pltpu.VMEM((NCHH, CH), jnp.int32),
        pltpu.VMEM((NCHH, CH), jnp.int32),
        pltpu.VMEM((2, CH, D), jnp.float32),
        pltpu.VMEM_SHARED((NPAD, D), jnp.float32),
        pltpu.SemaphoreType.DMA,
        pltpu.SemaphoreType.DMA,
        pltpu.SemaphoreType.DMA,
    ],
)(_conv_body)


BR = 2000
GRID = N // BR

def _row_spec(width):
    return pl.BlockSpec((BR, width), lambda i: (i, 0))

def _full_spec(shape):
    return pl.BlockSpec(shape, lambda i: (0,) * len(shape))


def _scale_body(x_ref, we_ref, be_ref, wg_ref, p0_ref, p1_ref,
                dinv_ref, g_ref):
    h = jnp.dot(x_ref[...], we_ref[...], preferred_element_type=jnp.float32)
    h = jnp.maximum(h + be_ref[...], 0.0)
    hr = jnp.dot(h, wg_ref[...], preferred_element_type=jnp.float32)
    deg = p0_ref[...] + p1_ref[...] + 1.0
    dinv = 1.0 / jnp.sqrt(deg)
    dinv_ref[...] = dinv
    g_ref[...] = dinv * hr


def _scale_call(x, W_enc, b_enc, W_gcn1, p0, p1):
    return pl.pallas_call(
        _scale_body,
        grid=(GRID,),
        in_specs=[_row_spec(D), _full_spec((D, D)), _full_spec((1, D)),
                  _full_spec((D, D)), _row_spec(D), _row_spec(D)],
        out_specs=[_row_spec(D), _row_spec(D)],
        out_shape=[jax.ShapeDtypeStruct((N, D), jnp.float32),
                   jax.ShapeDtypeStruct((N, D), jnp.float32)],
    )(x, W_enc, b_enc, W_gcn1, p0, p1)


def _mid_body(a0_ref, a1_ref, g_ref, dinv_ref, bg_ref, wu_ref, bu_ref,
              wn_ref, o_ref):
    dinv = dinv_ref[...]
    t = dinv * (a0_ref[...] + a1_ref[...] + g_ref[...]) + bg_ref[...]
    h = jnp.dot(t, wu_ref[...], preferred_element_type=jnp.float32)
    h = jnp.maximum(h + bu_ref[...], 0.0)
    o_ref[...] = dinv * jnp.dot(h, wn_ref[...],
                                preferred_element_type=jnp.float32)


def _mid_call(a0, a1, g, dinv, b_gcn, W_upd, b_upd, W_next):
    return pl.pallas_call(
        _mid_body,
        grid=(GRID,),
        in_specs=[_row_spec(D), _row_spec(D), _row_spec(D), _row_spec(D),
                  _full_spec((1, D)), _full_spec((D, D)), _full_spec((1, D)),
                  _full_spec((D, D))],
        out_specs=_row_spec(D),
        out_shape=jax.ShapeDtypeStruct((N, D), jnp.float32),
    )(a0, a1, g, dinv, b_gcn, W_upd, b_upd, W_next)


def _out_body(a0_ref, a1_ref, g_ref, dinv_ref, bg_ref, wu_ref, bu_ref, o_ref):
    dinv = dinv_ref[...]
    t = dinv * (a0_ref[...] + a1_ref[...] + g_ref[...]) + bg_ref[...]
    h = jnp.dot(t, wu_ref[...], preferred_element_type=jnp.float32)
    o_ref[...] = jnp.maximum(h + bu_ref[...], 0.0)


def _out_call(a0, a1, g, dinv, b_gcn, W_upd, b_upd):
    return pl.pallas_call(
        _out_body,
        grid=(GRID,),
        in_specs=[_row_spec(D), _row_spec(D), _row_spec(D), _row_spec(D),
                  _full_spec((1, D)), _full_spec((D, D)), _full_spec((1, D))],
        out_specs=_row_spec(D),
        out_shape=jax.ShapeDtypeStruct((N, D), jnp.float32),
    )(a0, a1, g, dinv, b_gcn, W_upd, b_upd)


def kernel(x, W_enc, b_enc, W_gcn1, b_gcn1, W_upd1, b_upd1,
           W_gcn2, b_gcn2, W_upd2, b_upd2, edge_index):
    src3d = edge_index[0].astype(jnp.int32).reshape(NW * NHALF, NCHH, CH)
    dst3d = edge_index[1].astype(jnp.int32).reshape(NW * NHALF, NCHH, CH)
    b_enc2 = b_enc.reshape(1, D)
    b_gcn1_2 = b_gcn1.reshape(1, D)
    b_upd1_2 = b_upd1.reshape(1, D)
    b_gcn2_2 = b_gcn2.reshape(1, D)
    b_upd2_2 = b_upd2.reshape(1, D)

    degp = _deg_call(dst3d)
    dinv, g1 = _scale_call(x, W_enc, b_enc2, W_gcn1, degp[0], degp[1])
    acc1 = _conv_call(g1, src3d, dst3d)
    g2 = _mid_call(acc1[0], acc1[1], g1, dinv, b_gcn1_2, W_upd1, b_upd1_2,
                   W_gcn2)
    acc2 = _conv_call(g2, src3d, dst3d)
    return _out_call(acc2[0], acc2[1], g2, dinv, b_gcn2_2, W_upd2, b_upd2_2)


# speedup vs baseline: 1.0247x; 1.0090x over previous
import functools

import jax
import jax.numpy as jnp
from jax import lax
from jax.experimental import pallas as pl
from jax.experimental.pallas import tpu as pltpu
from jax.experimental.pallas import tpu_sc as plsc

N = 10000
D = 128
E = 320000

NC = 2
NS = 16
NW = NC * NS

NPAD = 10240
RPT = NPAD // NS
EPT = E // NW
CH = 100
NCHT = EPT // CH
NHALF = 2
NCHH = NCHT // NHALF

_mesh = plsc.VectorSubcoreMesh(
    core_axis_name="c", subcore_axis_name="s", num_cores=NC, num_subcores=NS)

def _fill2d(ref, nrows, ncols, val):
    vec = jnp.full((16,), val, jnp.float32)
    def row(r, carry):
        for j in range(ncols // 16):
            ref[r, pl.ds(j * 16, 16)] = vec
        return carry
    lax.fori_loop(0, nrows, row, 0)


def _deg_body(dst_hbm, out_hbm, dst_v, ones_v, deg_sh, sem):
    c = lax.axis_index("c")
    s = lax.axis_index("s")
    wid = s * NC + c

    _fill2d(ones_v, CH, D, 0.0)
    r0 = s * RPT
    for k in range(RPT // CH):
        pltpu.sync_copy(ones_v, deg_sh.at[pl.ds(r0 + k * CH, CH)])
    rem = RPT - (RPT // CH) * CH
    if rem:
        pltpu.sync_copy(ones_v.at[pl.ds(0, rem)],
                        deg_sh.at[pl.ds(r0 + (RPT // CH) * CH, rem)])
    _fill2d(ones_v, CH, D, 1.0)
    plsc.subcore_barrier()

    def issue(j, carry):
        pltpu.async_copy(ones_v, deg_sh.at[dst_v.at[j]], sem, add=True)
        return carry
    def drain(j, carry):
        pltpu.make_async_copy(ones_v, deg_sh.at[dst_v.at[0]], sem).wait()
        return carry
    for half in range(NHALF):
        pltpu.sync_copy(dst_hbm.at[wid * NHALF + half], dst_v)
        lax.fori_loop(0, NCHH, issue, 0)
        lax.fori_loop(0, NCHH, drain, 0)

    plsc.subcore_barrier()
    pltpu.sync_copy(deg_sh.at[pl.ds(r0, RPT)], out_hbm.at[c, pl.ds(r0, RPT)])


_deg_call = functools.partial(
    pl.kernel,
    out_type=jax.ShapeDtypeStruct((NC, NPAD, D), jnp.float32),
    mesh=_mesh,
    scratch_types=[
        pltpu.VMEM((NCHH, CH), jnp.int32),
        pltpu.VMEM((CH, D), jnp.float32),
        pltpu.VMEM_SHARED((NPAD, D), jnp.float32),
        pltpu.SemaphoreType.DMA,
    ],
)(_deg_body)


def _conv_body(g_hbm, src_hbm, dst_hbm, out_hbm,
               src_v, dst_v, rows_v, acc_sh, gsem0, gsem1, ssem):
    c = lax.axis_index("c")
    s = lax.axis_index("s")
    wid = s * NC + c

    zvec = jnp.zeros((16,), jnp.float32)
    def zrow(r, carry):
        for j in range(D // 16):
            rows_v[0, r, pl.ds(j * 16, 16)] = zvec
        return carry
    lax.fori_loop(0, CH, zrow, 0)
    r0 = s * RPT
    for k in range(RPT // CH):
        pltpu.sync_copy(rows_v.at[0], acc_sh.at[pl.ds(r0 + k * CH, CH)])
    rem = RPT - (RPT // CH) * CH
    if rem:
        pltpu.sync_copy(rows_v.at[0].at[pl.ds(0, rem)],
                        acc_sh.at[pl.ds(r0 + (RPT // CH) * CH, rem)])
    plsc.subcore_barrier()

    def body(j2, carry):
        j = j2 * 2
        cpb = pltpu.async_copy(g_hbm.at[src_v.at[j + 1]], rows_v.at[1], gsem1)
        pltpu.make_async_copy(g_hbm.at[src_v.at[j]], rows_v.at[0],
                              gsem0).wait()
        pltpu.async_copy(rows_v.at[0], acc_sh.at[dst_v.at[j]], ssem,
                         add=True).wait()
        @pl.when(j2 < NCHH // 2 - 1)
        def _():
            pltpu.async_copy(g_hbm.at[src_v.at[j + 2]], rows_v.at[0], gsem0)
        cpb.wait()
        pltpu.async_copy(rows_v.at[1], acc_sh.at[dst_v.at[j + 1]], ssem,
                         add=True).wait()
        return carry
    for half in range(NHALF):
        pltpu.sync_copy(src_hbm.at[wid * NHALF + half], src_v)
        pltpu.sync_copy(dst_hbm.at[wid * NHALF + half], dst_v)
        pltpu.async_copy(g_hbm.at[src_v.at[0]], rows_v.at[0], gsem0)
        lax.fori_loop(0, NCHH // 2, body, 0)

    plsc.subcore_barrier()
    pltpu.sync_copy(acc_sh.at[pl.ds(r0, RPT)], out_hbm.at[c, pl.ds(r0, RPT)])


_conv_call = functools.partial(
    pl.kernel,
    out_type=jax.ShapeDtypeStruct((NC, NPAD, D), jnp.float32),
    mesh=_mesh,
    scratch_types=[
        pltpu.VMEM((NCHH, CH), jnp.int32),
        pltpu.VMEM((NCHH, CH), jnp.int32),
        pltpu.VMEM((2, CH, D), jnp.float32),
        pltpu.VMEM_SHARED((NPAD, D), jnp.float32),
        pltpu.SemaphoreType.DMA,
        pltpu.SemaphoreType.DMA,
        pltpu.SemaphoreType.DMA,
    ],
)(_conv_body)


BR = 5000
GRID = N // BR

def _row_spec(width):
    return pl.BlockSpec((BR, width), lambda i: (i, 0))

def _full_spec(shape):
    return pl.BlockSpec(shape, lambda i: (0,) * len(shape))


def _scale_body(x_ref, we_ref, be_ref, wg_ref, p0_ref, p1_ref,
                dinv_ref, g_ref):
    h = jnp.dot(x_ref[...], we_ref[...], preferred_element_type=jnp.float32)
    h = jnp.maximum(h + be_ref[...], 0.0)
    hr = jnp.dot(h, wg_ref[...], preferred_element_type=jnp.float32)
    deg = p0_ref[...] + p1_ref[...] + 1.0
    dinv = 1.0 / jnp.sqrt(deg)
    dinv_ref[...] = dinv
    g_ref[...] = dinv * hr


def _scale_call(x, W_enc, b_enc, W_gcn1, p0, p1):
    return pl.pallas_call(
        _scale_body,
        grid=(GRID,),
        in_specs=[_row_spec(D), _full_spec((D, D)), _full_spec((1, D)),
                  _full_spec((D, D)), _row_spec(D), _row_spec(D)],
        out_specs=[_row_spec(D), _row_spec(D)],
        out_shape=[jax.ShapeDtypeStruct((N, D), jnp.float32),
                   jax.ShapeDtypeStruct((N, D), jnp.float32)],
    )(x, W_enc, b_enc, W_gcn1, p0, p1)


def _mid_body(a0_ref, a1_ref, g_ref, dinv_ref, bg_ref, wu_ref, bu_ref,
              wn_ref, o_ref):
    dinv = dinv_ref[...]
    t = dinv * (a0_ref[...] + a1_ref[...] + g_ref[...]) + bg_ref[...]
    h = jnp.dot(t, wu_ref[...], preferred_element_type=jnp.float32)
    h = jnp.maximum(h + bu_ref[...], 0.0)
    o_ref[...] = dinv * jnp.dot(h, wn_ref[...],
                                preferred_element_type=jnp.float32)


def _mid_call(a0, a1, g, dinv, b_gcn, W_upd, b_upd, W_next):
    return pl.pallas_call(
        _mid_body,
        grid=(GRID,),
        in_specs=[_row_spec(D), _row_spec(D), _row_spec(D), _row_spec(D),
                  _full_spec((1, D)), _full_spec((D, D)), _full_spec((1, D)),
                  _full_spec((D, D))],
        out_specs=_row_spec(D),
        out_shape=jax.ShapeDtypeStruct((N, D), jnp.float32),
    )(a0, a1, g, dinv, b_gcn, W_upd, b_upd, W_next)


def _out_body(a0_ref, a1_ref, g_ref, dinv_ref, bg_ref, wu_ref, bu_ref, o_ref):
    dinv = dinv_ref[...]
    t = dinv * (a0_ref[...] + a1_ref[...] + g_ref[...]) + bg_ref[...]
    h = jnp.dot(t, wu_ref[...], preferred_element_type=jnp.float32)
    o_ref[...] = jnp.maximum(h + bu_ref[...], 0.0)


def _out_call(a0, a1, g, dinv, b_gcn, W_upd, b_upd):
    return pl.pallas_call(
        _out_body,
        grid=(GRID,),
        in_specs=[_row_spec(D), _row_spec(D), _row_spec(D), _row_spec(D),
                  _full_spec((1, D)), _full_spec((D, D)), _full_spec((1, D))],
        out_specs=_row_spec(D),
        out_shape=jax.ShapeDtypeStruct((N, D), jnp.float32),
    )(a0, a1, g, dinv, b_gcn, W_upd, b_upd)


def kernel(x, W_enc, b_enc, W_gcn1, b_gcn1, W_upd1, b_upd1,
           W_gcn2, b_gcn2, W_upd2, b_upd2, edge_index):
    src3d = edge_index[0].astype(jnp.int32).reshape(NW * NHALF, NCHH, CH)
    dst3d = edge_index[1].astype(jnp.int32).reshape(NW * NHALF, NCHH, CH)
    b_enc2 = b_enc.reshape(1, D)
    b_gcn1_2 = b_gcn1.reshape(1, D)
    b_upd1_2 = b_upd1.reshape(1, D)
    b_gcn2_2 = b_gcn2.reshape(1, D)
    b_upd2_2 = b_upd2.reshape(1, D)

    degp = _deg_call(dst3d)
    dinv, g1 = _scale_call(x, W_enc, b_enc2, W_gcn1, degp[0], degp[1])
    acc1 = _conv_call(g1, src3d, dst3d)
    g2 = _mid_call(acc1[0], acc1[1], g1, dinv, b_gcn1_2, W_upd1, b_upd1_2,
                   W_gcn2)
    acc2 = _conv_call(g2, src3d, dst3d)
    return _out_call(acc2[0], acc2[1], g2, dinv, b_gcn2_2, W_upd2, b_upd2_2)
